# Initial kernel scaffold; baseline (speedup 1.0000x reference)
import jax
import jax.numpy as jnp
from jax.experimental import pallas as pl


def kernel(x, edge_index1, edge_index2, pos_src, pos_dst, neg_src, neg_dst, W_self1, W_neigh1, b1, W_self2, W_neigh2, b2, Wp1, bp1, Wp2, bp2, Wp3, bp3):
    raise NotImplementedError("write your pallas kernel here")


# speedup vs baseline: 6.1458x; 6.1458x over previous
import functools

import jax
import jax.numpy as jnp
from jax import lax
from jax.experimental import pallas as pl
from jax.experimental.pallas import tpu as pltpu
from jax.experimental.pallas import tpu_sc as plsc

N = 10000
D = 128
H = 128
E = 320000
P = 65536
NC = 2
NS = 16
NW = NC * NS
EPW = E // NW
C = 80
CH = EPW // C
NPAD = 10240
NPT = NPAD // NS
PP = 2 * P
PPW = PP // NW
PC = 128
PCH = PPW // PC

f32 = jnp.float32


def _sc_aggregate(table, src_r, dst_r, zeros2d, zeros1d):
    mesh = plsc.VectorSubcoreMesh(core_axis_name="c", subcore_axis_name="s")

    @functools.partial(
        pl.kernel,
        out_type=(jax.ShapeDtypeStruct((NC, NPAD, D), f32),
                  jax.ShapeDtypeStruct((NC, 1, NPAD), f32)),
        mesh=mesh,
        scratch_types=[
            pltpu.VMEM((CH, 1, C), jnp.int32),
            pltpu.VMEM((CH, 1, C), jnp.int32),
            pltpu.VMEM((C, D), f32),
            pltpu.VMEM((C,), f32),
            pltpu.VMEM_SHARED((NPAD, D), f32),
            pltpu.VMEM_SHARED((NPAD,), f32),
            pltpu.SemaphoreType.DMA,
        ],
    )
    def k(tab_hbm, src_hbm, dst_hbm, z2_hbm, z1_hbm, acc_out, deg_out,
          src_v, dst_v, rows_v, ones_v, acc_sh, deg_sh, sem):
        cid = lax.axis_index("c")
        sid = lax.axis_index("s")
        wid = cid * NS + sid
        off = pl.multiple_of(sid * NPT, 128)
        pltpu.sync_copy(z2_hbm.at[pl.ds(off, NPT)], acc_sh.at[pl.ds(off, NPT)])
        pltpu.sync_copy(z1_hbm.at[pl.ds(off, NPT)], deg_sh.at[pl.ds(off, NPT)])
        pltpu.sync_copy(src_hbm.at[wid], src_v)
        pltpu.sync_copy(dst_hbm.at[wid], dst_v)

        ones16 = jnp.ones((16,), f32)
        for g in range(C // 16):
            ones_v[pl.ds(g * 16, 16)] = ones16
        plsc.subcore_barrier()

        def chunk(j, carry):
            pltpu.async_copy(tab_hbm.at[src_v.at[j, 0]], rows_v, sem).wait()
            pltpu.sync_copy(rows_v, acc_sh.at[dst_v.at[j, 0]], add=True)
            pltpu.sync_copy(ones_v, deg_sh.at[dst_v.at[j, 0]], add=True)
            return carry

        lax.fori_loop(0, CH, chunk, 0)
        plsc.subcore_barrier()
        pltpu.sync_copy(acc_sh.at[pl.ds(off, NPT)],
                        acc_out.at[cid, pl.ds(off, NPT)])
        pltpu.sync_copy(deg_sh.at[pl.ds(off, NPT)],
                        deg_out.at[cid, 0, pl.ds(off, NPT)])

    return k(table, src_r, dst_r, zeros2d, zeros1d)


def _tc_combine1(x, acc, deg_t, Ws, Wn, br):
    R = 2000

    def body(x_ref, acc_ref, deg_ref, ws_ref, wn_ref, b_ref, out_ref):
        accs = acc_ref[0] + acc_ref[1]
        deg = jnp.maximum(jnp.sum(deg_ref[:], axis=1, keepdims=True), 1.0)
        mean = accs / deg
        h = (jnp.dot(x_ref[:], ws_ref[:], preferred_element_type=f32)
             + jnp.dot(mean, wn_ref[:], preferred_element_type=f32)
             + b_ref[:])
        out_ref[:] = jnp.maximum(h, 0.0)

    return pl.pallas_call(
        body,
        grid=(N // R,),
        in_specs=[
            pl.BlockSpec((R, D), lambda i: (i, 0)),
            pl.BlockSpec((NC, R, D), lambda i: (0, i, 0)),
            pl.BlockSpec((R, NC), lambda i: (i, 0)),
            pl.BlockSpec((D, H), lambda i: (0, 0)),
            pl.BlockSpec((D, H), lambda i: (0, 0)),
            pl.BlockSpec((1, H), lambda i: (0, 0)),
        ],
        out_specs=pl.BlockSpec((R, D), lambda i: (i, 0)),
        out_shape=jax.ShapeDtypeStruct((N, D), f32),
    )(x, acc, deg_t, Ws, Wn, br)


def _tc_combine2(h1, acc, deg_t, Ws, Wn, br, Wu, Wv):
    R = 2000

    def body(h_ref, acc_ref, deg_ref, ws_ref, wn_ref, b_ref, wu_ref, wv_ref,
             u_ref, v_ref):
        accs = acc_ref[0] + acc_ref[1]
        deg = jnp.maximum(jnp.sum(deg_ref[:], axis=1, keepdims=True), 1.0)
        mean = accs / deg
        h2 = (jnp.dot(h_ref[:], ws_ref[:], preferred_element_type=f32)
              + jnp.dot(mean, wn_ref[:], preferred_element_type=f32)
              + b_ref[:])
        u_ref[:] = jnp.dot(h2, wu_ref[:], preferred_element_type=f32)
        v_ref[:] = jnp.dot(h2, wv_ref[:], preferred_element_type=f32)

    return pl.pallas_call(
        body,
        grid=(N // R,),
        in_specs=[
            pl.BlockSpec((R, D), lambda i: (i, 0)),
            pl.BlockSpec((NC, R, D), lambda i: (0, i, 0)),
            pl.BlockSpec((R, NC), lambda i: (i, 0)),
            pl.BlockSpec((H, H), lambda i: (0, 0)),
            pl.BlockSpec((H, H), lambda i: (0, 0)),
            pl.BlockSpec((1, H), lambda i: (0, 0)),
            pl.BlockSpec((H, H), lambda i: (0, 0)),
            pl.BlockSpec((H, H), lambda i: (0, 0)),
        ],
        out_specs=[
            pl.BlockSpec((R, H), lambda i: (i, 0)),
            pl.BlockSpec((R, H), lambda i: (i, 0)),
        ],
        out_shape=[
            jax.ShapeDtypeStruct((N, H), f32),
            jax.ShapeDtypeStruct((N, H), f32),
        ],
    )(h1, acc, deg_t, Ws, Wn, br, Wu, Wv)


def _sc_pair_gather(u, v, src_r, dst_r):
    mesh = plsc.VectorSubcoreMesh(core_axis_name="c", subcore_axis_name="s")

    @functools.partial(
        pl.kernel,
        out_type=(jax.ShapeDtypeStruct((PP, H), f32),
                  jax.ShapeDtypeStruct((PP, H), f32)),
        mesh=mesh,
        scratch_types=[
            pltpu.VMEM((PCH, 1, PC), jnp.int32),
            pltpu.VMEM((PCH, 1, PC), jnp.int32),
            pltpu.VMEM((PC, H), f32),
            pltpu.VMEM((PC, H), f32),
            pltpu.SemaphoreType.DMA,
            pltpu.SemaphoreType.DMA,
        ],
    )
    def k(u_hbm, v_hbm, src_hbm, dst_hbm, gu_out, gv_out, src_v, dst_v,
          urows_v, vrows_v, usem, vsem):
        cid = lax.axis_index("c")
        sid = lax.axis_index("s")
        wid = cid * NS + sid
        pltpu.sync_copy(src_hbm.at[wid], src_v)
        pltpu.sync_copy(dst_hbm.at[wid], dst_v)
        base = wid * PPW

        def chunk(j, carry):
            out_off = pl.multiple_of(base + j * PC, 8)
            ucp = pltpu.async_copy(u_hbm.at[src_v.at[j, 0]], urows_v, usem)
            vcp = pltpu.async_copy(v_hbm.at[dst_v.at[j, 0]], vrows_v, vsem)
            ucp.wait()
            pltpu.sync_copy(urows_v, gu_out.at[pl.ds(out_off, PC)])
            vcp.wait()
            pltpu.sync_copy(vrows_v, gv_out.at[pl.ds(out_off, PC)])
            return carry

        lax.fori_loop(0, PCH, chunk, 0)

    return k(u, v, src_r, dst_r)


def _tc_mlp(gu, gv, b1r, W2, b2r, W3, b3r):
    R = 4096

    def body(gu_ref, gv_ref, b1_ref, w2_ref, b2_ref, w3_ref, b3_ref, out_ref):
        z1 = jnp.maximum(gu_ref[:] + gv_ref[:] + b1_ref[:], 0.0)
        z2 = jnp.maximum(
            jnp.dot(z1, w2_ref[:], preferred_element_type=f32) + b2_ref[:],
            0.0)
        out_ref[:] = jnp.sum(z2 * w3_ref[:], axis=1, keepdims=True) + b3_ref[:]

    return pl.pallas_call(
        body,
        grid=(PP // R,),
        in_specs=[
            pl.BlockSpec((R, H), lambda i: (i, 0)),
            pl.BlockSpec((R, H), lambda i: (i, 0)),
            pl.BlockSpec((1, H), lambda i: (0, 0)),
            pl.BlockSpec((H, H), lambda i: (0, 0)),
            pl.BlockSpec((1, H), lambda i: (0, 0)),
            pl.BlockSpec((1, H), lambda i: (0, 0)),
            pl.BlockSpec((1, 1), lambda i: (0, 0)),
        ],
        out_specs=pl.BlockSpec((R, 1), lambda i: (i, 0)),
        out_shape=jax.ShapeDtypeStruct((PP, 1), f32),
    )(gu, gv, b1r, W2, b2r, W3, b3r)


def kernel(x, edge_index1, edge_index2, pos_src, pos_dst, neg_src, neg_dst,
           W_self1, W_neigh1, b1, W_self2, W_neigh2, b2,
           Wp1, bp1, Wp2, bp2, Wp3, bp3):
    zeros2d = jnp.zeros((NPAD, D), f32)
    zeros1d = jnp.zeros((NPAD,), f32)
    src1 = edge_index1[0].reshape(NW, CH, 1, C)
    dst1 = edge_index1[1].reshape(NW, CH, 1, C)
    src2 = edge_index2[0].reshape(NW, CH, 1, C)
    dst2 = edge_index2[1].reshape(NW, CH, 1, C)

    acc1, deg1 = _sc_aggregate(x, src1, dst1, zeros2d, zeros1d)
    h1 = _tc_combine1(x, acc1, deg1.reshape(NC, NPAD).T,
                      W_self1, W_neigh1, b1.reshape(1, H))
    acc2, deg2 = _sc_aggregate(h1, src2, dst2, zeros2d, zeros1d)
    u, v = _tc_combine2(h1, acc2, deg2.reshape(NC, NPAD).T,
                        W_self2, W_neigh2, b2.reshape(1, H),
                        Wp1[:H], Wp1[H:])

    all_src = jnp.concatenate([pos_src, neg_src]).reshape(NW, PCH, 1, PC)
    all_dst = jnp.concatenate([pos_dst, neg_dst]).reshape(NW, PCH, 1, PC)
    gu, gv = _sc_pair_gather(u, v, all_src, all_dst)

    scores = _tc_mlp(gu, gv, bp1.reshape(1, H), Wp2, bp2.reshape(1, H),
                     Wp3.reshape(1, H), bp3.reshape(1, 1))
    return scores[:P], scores[P:]


# speedup vs baseline: 8.5062x; 1.3841x over previous
import functools

import jax
import jax.numpy as jnp
from jax import lax
from jax.experimental import pallas as pl
from jax.experimental.pallas import tpu as pltpu
from jax.experimental.pallas import tpu_sc as plsc

N = 10000
D = 128
H = 128
E = 320000
P = 65536
NC = 2
NS = 16
NW = NC * NS
C = 80
CH = 126
PH = CH // 2
EPAD = NW * CH * C - E
NPAD = 10240
NPT = NPAD // NS
PP = 2 * P
PPW = PP // NW
PC = 128
PCH = PPW // PC

f32 = jnp.float32


def _sc_aggregate(table, src_r, dst_r, zeros2d, zeros1d):
    mesh = plsc.VectorSubcoreMesh(core_axis_name="c", subcore_axis_name="s")

    @functools.partial(
        pl.kernel,
        out_type=(jax.ShapeDtypeStruct((NC, NPAD, D), f32),
                  jax.ShapeDtypeStruct((NC, 1, NPAD), f32)),
        mesh=mesh,
        scratch_types=[
            pltpu.VMEM((PH, 1, C), jnp.int32),
            pltpu.VMEM((PH, 1, C), jnp.int32),
            pltpu.VMEM((C, D), f32),
            pltpu.VMEM((C, D), f32),
            pltpu.VMEM((C,), f32),
            pltpu.VMEM_SHARED((NPAD, D), f32),
            pltpu.VMEM_SHARED((NPAD,), f32),
            pltpu.SemaphoreType.DMA,
            pltpu.SemaphoreType.DMA,
        ],
    )
    def k(tab_hbm, src_hbm, dst_hbm, z2_hbm, z1_hbm, acc_out, deg_out,
          src_v, dst_v, rows0_v, rows1_v, ones_v, acc_sh, deg_sh, sem0, sem1):
        cid = lax.axis_index("c")
        sid = lax.axis_index("s")
        wid = cid * NS + sid
        off = pl.multiple_of(sid * NPT, 128)
        pltpu.sync_copy(z2_hbm.at[pl.ds(off, NPT)], acc_sh.at[pl.ds(off, NPT)])
        pltpu.sync_copy(z1_hbm.at[pl.ds(off, NPT)], deg_sh.at[pl.ds(off, NPT)])

        ones16 = jnp.ones((16,), f32)
        for g in range(C // 16):
            ones_v[pl.ds(g * 16, 16)] = ones16
        plsc.subcore_barrier()

        def start(j, buf, sem):
            pltpu.async_copy(tab_hbm.at[src_v.at[j, 0]], buf, sem)

        def wait(j, buf, sem):
            pltpu.make_async_copy(tab_hbm.at[src_v.at[j, 0]], buf, sem).wait()

        def drain(j, buf):
            pltpu.sync_copy(buf, acc_sh.at[dst_v.at[j, 0]], add=True)
            pltpu.sync_copy(ones_v, deg_sh.at[dst_v.at[j, 0]], add=True)

        for p in range(2):
            pltpu.sync_copy(src_hbm.at[wid, pl.ds(p * PH, PH)], src_v)
            pltpu.sync_copy(dst_hbm.at[wid, pl.ds(p * PH, PH)], dst_v)
            start(0, rows0_v, sem0)

            def chunk2(jj, carry):
                j0 = jj * 2
                j1 = j0 + 1
                start(j1, rows1_v, sem1)
                wait(j0, rows0_v, sem0)
                drain(j0, rows0_v)
                start(j1 + 1, rows0_v, sem0)
                wait(j1, rows1_v, sem1)
                drain(j1, rows1_v)
                return carry

            lax.fori_loop(0, PH // 2, chunk2, 0)
            wait(PH - 1, rows0_v, sem0)
            drain(PH - 1, rows0_v)
        plsc.subcore_barrier()
        pltpu.sync_copy(acc_sh.at[pl.ds(off, NPT)],
                        acc_out.at[cid, pl.ds(off, NPT)])
        pltpu.sync_copy(deg_sh.at[pl.ds(off, NPT)],
                        deg_out.at[cid, 0, pl.ds(off, NPT)])

    return k(table, src_r, dst_r, zeros2d, zeros1d)


def _tc_combine1(x, acc, deg_t, Ws, Wn, br):
    R = 2000

    def body(x_ref, acc_ref, deg_ref, ws_ref, wn_ref, b_ref, out_ref):
        accs = acc_ref[0] + acc_ref[1]
        deg = jnp.maximum(jnp.sum(deg_ref[:], axis=1, keepdims=True), 1.0)
        mean = accs / deg
        h = (jnp.dot(x_ref[:], ws_ref[:], preferred_element_type=f32)
             + jnp.dot(mean, wn_ref[:], preferred_element_type=f32)
             + b_ref[:])
        out_ref[:] = jnp.maximum(h, 0.0)

    return pl.pallas_call(
        body,
        grid=(N // R,),
        in_specs=[
            pl.BlockSpec((R, D), lambda i: (i, 0)),
            pl.BlockSpec((NC, R, D), lambda i: (0, i, 0)),
            pl.BlockSpec((R, NC), lambda i: (i, 0)),
            pl.BlockSpec((D, H), lambda i: (0, 0)),
            pl.BlockSpec((D, H), lambda i: (0, 0)),
            pl.BlockSpec((1, H), lambda i: (0, 0)),
        ],
        out_specs=pl.BlockSpec((R, D), lambda i: (i, 0)),
        out_shape=jax.ShapeDtypeStruct((N, D), f32),
    )(x, acc, deg_t, Ws, Wn, br)


def _tc_combine2(h1, acc, deg_t, Ws, Wn, br, Wu, Wv):
    R = 2000

    def body(h_ref, acc_ref, deg_ref, ws_ref, wn_ref, b_ref, wu_ref, wv_ref,
             u_ref, v_ref):
        accs = acc_ref[0] + acc_ref[1]
        deg = jnp.maximum(jnp.sum(deg_ref[:], axis=1, keepdims=True), 1.0)
        mean = accs / deg
        h2 = (jnp.dot(h_ref[:], ws_ref[:], preferred_element_type=f32)
              + jnp.dot(mean, wn_ref[:], preferred_element_type=f32)
              + b_ref[:])
        u_ref[:] = jnp.dot(h2, wu_ref[:], preferred_element_type=f32)
        v_ref[:] = jnp.dot(h2, wv_ref[:], preferred_element_type=f32)

    return pl.pallas_call(
        body,
        grid=(N // R,),
        in_specs=[
            pl.BlockSpec((R, D), lambda i: (i, 0)),
            pl.BlockSpec((NC, R, D), lambda i: (0, i, 0)),
            pl.BlockSpec((R, NC), lambda i: (i, 0)),
            pl.BlockSpec((H, H), lambda i: (0, 0)),
            pl.BlockSpec((H, H), lambda i: (0, 0)),
            pl.BlockSpec((1, H), lambda i: (0, 0)),
            pl.BlockSpec((H, H), lambda i: (0, 0)),
            pl.BlockSpec((H, H), lambda i: (0, 0)),
        ],
        out_specs=[
            pl.BlockSpec((R, H), lambda i: (i, 0)),
            pl.BlockSpec((R, H), lambda i: (i, 0)),
        ],
        out_shape=[
            jax.ShapeDtypeStruct((N, H), f32),
            jax.ShapeDtypeStruct((N, H), f32),
        ],
    )(h1, acc, deg_t, Ws, Wn, br, Wu, Wv)


def _sc_pair_gather_add(u, v, src_r, dst_r):
    mesh = plsc.VectorSubcoreMesh(core_axis_name="c", subcore_axis_name="s")

    @functools.partial(
        pl.kernel,
        out_type=jax.ShapeDtypeStruct((PP, H), f32),
        mesh=mesh,
        scratch_types=[
            pltpu.VMEM((PCH, 1, PC), jnp.int32),
            pltpu.VMEM((PCH, 1, PC), jnp.int32),
            pltpu.VMEM((PC, H), f32),
            pltpu.VMEM((PC, H), f32),
            pltpu.VMEM((PC, H), f32),
            pltpu.VMEM((PC, H), f32),
            pltpu.SemaphoreType.DMA,
            pltpu.SemaphoreType.DMA,
            pltpu.SemaphoreType.DMA,
            pltpu.SemaphoreType.DMA,
        ],
    )
    def k(u_hbm, v_hbm, src_hbm, dst_hbm, z_out, src_v, dst_v,
          u0_v, v0_v, u1_v, v1_v, us0, vs0, us1, vs1):
        cid = lax.axis_index("c")
        sid = lax.axis_index("s")
        wid = cid * NS + sid
        pltpu.sync_copy(src_hbm.at[wid], src_v)
        pltpu.sync_copy(dst_hbm.at[wid], dst_v)
        base = wid * PPW

        def start(j, ub, vb, us, vs):
            pltpu.async_copy(u_hbm.at[src_v.at[j, 0]], ub, us)
            pltpu.async_copy(v_hbm.at[dst_v.at[j, 0]], vb, vs)

        def wait(j, ub, vb, us, vs):
            pltpu.make_async_copy(u_hbm.at[src_v.at[j, 0]], ub, us).wait()
            pltpu.make_async_copy(v_hbm.at[dst_v.at[j, 0]], vb, vs).wait()

        def drain(j, ub, vb):
            def addrow(i, carry):
                for g in range(H // 16):
                    sl = pl.ds(g * 16, 16)
                    ub[i, sl] = ub[i, sl] + vb[i, sl]
                return carry

            lax.fori_loop(0, PC, addrow, 0)
            out_off = pl.multiple_of(base + j * PC, 8)
            pltpu.sync_copy(ub, z_out.at[pl.ds(out_off, PC)])

        start(0, u0_v, v0_v, us0, vs0)

        def chunk2(jj, carry):
            j0 = jj * 2
            j1 = j0 + 1
            start(j1, u1_v, v1_v, us1, vs1)
            wait(j0, u0_v, v0_v, us0, vs0)
            drain(j0, u0_v, v0_v)
            start(jnp.minimum(j1 + 1, PCH - 1), u0_v, v0_v, us0, vs0)
            wait(j1, u1_v, v1_v, us1, vs1)
            drain(j1, u1_v, v1_v)
            return carry

        lax.fori_loop(0, PCH // 2, chunk2, 0)
        wait(PCH - 1, u0_v, v0_v, us0, vs0)

    return k(u, v, src_r, dst_r)


def _tc_mlp(z, b1r, W2, b2r, W3, b3r):
    R = 4096

    def body(z_ref, b1_ref, w2_ref, b2_ref, w3_ref, b3_ref, out_ref):
        z1 = jnp.maximum(z_ref[:] + b1_ref[:], 0.0)
        z2 = jnp.maximum(
            jnp.dot(z1, w2_ref[:], preferred_element_type=f32) + b2_ref[:],
            0.0)
        out_ref[:] = jnp.sum(z2 * w3_ref[:], axis=1, keepdims=True) + b3_ref[:]

    return pl.pallas_call(
        body,
        grid=(PP // R,),
        in_specs=[
            pl.BlockSpec((R, H), lambda i: (i, 0)),
            pl.BlockSpec((1, H), lambda i: (0, 0)),
            pl.BlockSpec((H, H), lambda i: (0, 0)),
            pl.BlockSpec((1, H), lambda i: (0, 0)),
            pl.BlockSpec((1, H), lambda i: (0, 0)),
            pl.BlockSpec((1, 1), lambda i: (0, 0)),
        ],
        out_specs=pl.BlockSpec((R, 1), lambda i: (i, 0)),
        out_shape=jax.ShapeDtypeStruct((PP, 1), f32),
    )(z, b1r, W2, b2r, W3, b3r)


def kernel(x, edge_index1, edge_index2, pos_src, pos_dst, neg_src, neg_dst,
           W_self1, W_neigh1, b1, W_self2, W_neigh2, b2,
           Wp1, bp1, Wp2, bp2, Wp3, bp3):
    zeros2d = jnp.zeros((NPAD, D), f32)
    zeros1d = jnp.zeros((NPAD,), f32)
    pad_src = (jnp.arange(EPAD, dtype=jnp.int32) * 37) % N
    pad_dst = N + jnp.arange(EPAD, dtype=jnp.int32) % (NPAD - N)
    src1 = jnp.concatenate([edge_index1[0], pad_src]).reshape(NW, CH, 1, C)
    dst1 = jnp.concatenate([edge_index1[1], pad_dst]).reshape(NW, CH, 1, C)
    src2 = jnp.concatenate([edge_index2[0], pad_src]).reshape(NW, CH, 1, C)
    dst2 = jnp.concatenate([edge_index2[1], pad_dst]).reshape(NW, CH, 1, C)

    acc1, deg1 = _sc_aggregate(x, src1, dst1, zeros2d, zeros1d)
    h1 = _tc_combine1(x, acc1, deg1.reshape(NC, NPAD).T,
                      W_self1, W_neigh1, b1.reshape(1, H))
    acc2, deg2 = _sc_aggregate(h1, src2, dst2, zeros2d, zeros1d)
    u, v = _tc_combine2(h1, acc2, deg2.reshape(NC, NPAD).T,
                        W_self2, W_neigh2, b2.reshape(1, H),
                        Wp1[:H], Wp1[H:])

    all_src = jnp.concatenate([pos_src, neg_src]).reshape(NW, PCH, 1, PC)
    all_dst = jnp.concatenate([pos_dst, neg_dst]).reshape(NW, PCH, 1, PC)
    z = _sc_pair_gather_add(u, v, all_src, all_dst)

    scores = _tc_mlp(z, bp1.reshape(1, H), Wp2, bp2.reshape(1, H),
                     Wp3.reshape(1, H), bp3.reshape(1, 1))
    return scores[:P], scores[P:]


# speedup vs baseline: 9.0364x; 1.0623x over previous
import functools

import jax
import jax.numpy as jnp
from jax import lax
from jax.experimental import pallas as pl
from jax.experimental.pallas import tpu as pltpu
from jax.experimental.pallas import tpu_sc as plsc

N = 10000
D = 128
H = 128
E = 320000
P = 65536
NC = 2
NS = 16
NW = NC * NS
C = 128
CH = 80
PH = CH // 2
EPAD = NW * CH * C - E
NPAD = 10240
NPT = NPAD // NS
PP = 2 * P
PPW = PP // NW
PC = 128
PCH = PPW // PC

f32 = jnp.float32


def _sc_aggregate(table, src_r, dst_r, zeros2d, zeros1d):
    mesh = plsc.VectorSubcoreMesh(core_axis_name="c", subcore_axis_name="s")

    @functools.partial(
        pl.kernel,
        out_type=(jax.ShapeDtypeStruct((NC, NPAD, D), f32),
                  jax.ShapeDtypeStruct((NC, 1, NPAD), f32)),
        mesh=mesh,
        scratch_types=[
            pltpu.VMEM((PH, 1, C), jnp.int32),
            pltpu.VMEM((PH, 1, C), jnp.int32),
            pltpu.VMEM((C, D), f32),
            pltpu.VMEM((C, D), f32),
            pltpu.VMEM((C,), f32),
            pltpu.VMEM_SHARED((NPAD, D), f32),
            pltpu.VMEM_SHARED((NPAD,), f32),
            pltpu.SemaphoreType.DMA,
            pltpu.SemaphoreType.DMA,
        ],
    )
    def k(tab_hbm, src_hbm, dst_hbm, z2_hbm, z1_hbm, acc_out, deg_out,
          src_v, dst_v, rows0_v, rows1_v, ones_v, acc_sh, deg_sh, sem0, sem1):
        cid = lax.axis_index("c")
        sid = lax.axis_index("s")
        wid = cid * NS + sid
        off = pl.multiple_of(sid * NPT, 128)
        pltpu.sync_copy(z2_hbm.at[pl.ds(off, NPT)], acc_sh.at[pl.ds(off, NPT)])
        pltpu.sync_copy(z1_hbm.at[pl.ds(off, NPT)], deg_sh.at[pl.ds(off, NPT)])

        ones16 = jnp.ones((16,), f32)
        for g in range(C // 16):
            ones_v[pl.ds(g * 16, 16)] = ones16
        plsc.subcore_barrier()

        def start(j, buf, sem):
            pltpu.async_copy(tab_hbm.at[src_v.at[j, 0]], buf, sem)

        def wait(j, buf, sem):
            pltpu.make_async_copy(tab_hbm.at[src_v.at[j, 0]], buf, sem).wait()

        def drain(j, buf):
            pltpu.sync_copy(buf, acc_sh.at[dst_v.at[j, 0]], add=True)
            pltpu.sync_copy(ones_v, deg_sh.at[dst_v.at[j, 0]], add=True)

        for p in range(2):
            pltpu.sync_copy(src_hbm.at[wid, pl.ds(p * PH, PH)], src_v)
            pltpu.sync_copy(dst_hbm.at[wid, pl.ds(p * PH, PH)], dst_v)
            start(0, rows0_v, sem0)

            def chunk2(jj, carry):
                j0 = jj * 2
                j1 = j0 + 1
                start(j1, rows1_v, sem1)
                wait(j0, rows0_v, sem0)
                drain(j0, rows0_v)
                start(jnp.minimum(j1 + 1, PH - 1), rows0_v, sem0)
                wait(j1, rows1_v, sem1)
                drain(j1, rows1_v)
                return carry

            lax.fori_loop(0, PH // 2, chunk2, 0)
            wait(PH - 1, rows0_v, sem0)
        plsc.subcore_barrier()
        pltpu.sync_copy(acc_sh.at[pl.ds(off, NPT)],
                        acc_out.at[cid, pl.ds(off, NPT)])
        pltpu.sync_copy(deg_sh.at[pl.ds(off, NPT)],
                        deg_out.at[cid, 0, pl.ds(off, NPT)])

    return k(table, src_r, dst_r, zeros2d, zeros1d)


def _tc_combine1(x, acc, deg_t, Ws, Wn, br):
    R = 2000

    def body(x_ref, acc_ref, deg_ref, ws_ref, wn_ref, b_ref, out_ref):
        accs = acc_ref[0] + acc_ref[1]
        deg = jnp.maximum(jnp.sum(deg_ref[:], axis=1, keepdims=True), 1.0)
        mean = accs / deg
        h = (jnp.dot(x_ref[:], ws_ref[:], preferred_element_type=f32)
             + jnp.dot(mean, wn_ref[:], preferred_element_type=f32)
             + b_ref[:])
        out_ref[:] = jnp.maximum(h, 0.0)

    return pl.pallas_call(
        body,
        grid=(N // R,),
        in_specs=[
            pl.BlockSpec((R, D), lambda i: (i, 0)),
            pl.BlockSpec((NC, R, D), lambda i: (0, i, 0)),
            pl.BlockSpec((R, NC), lambda i: (i, 0)),
            pl.BlockSpec((D, H), lambda i: (0, 0)),
            pl.BlockSpec((D, H), lambda i: (0, 0)),
            pl.BlockSpec((1, H), lambda i: (0, 0)),
        ],
        out_specs=pl.BlockSpec((R, D), lambda i: (i, 0)),
        out_shape=jax.ShapeDtypeStruct((N, D), f32),
    )(x, acc, deg_t, Ws, Wn, br)


def _tc_combine2(h1, acc, deg_t, Ws, Wn, br, Wu, Wv):
    R = 2000

    def body(h_ref, acc_ref, deg_ref, ws_ref, wn_ref, b_ref, wu_ref, wv_ref,
             u_ref, v_ref):
        accs = acc_ref[0] + acc_ref[1]
        deg = jnp.maximum(jnp.sum(deg_ref[:], axis=1, keepdims=True), 1.0)
        mean = accs / deg
        h2 = (jnp.dot(h_ref[:], ws_ref[:], preferred_element_type=f32)
              + jnp.dot(mean, wn_ref[:], preferred_element_type=f32)
              + b_ref[:])
        u_ref[:] = jnp.dot(h2, wu_ref[:], preferred_element_type=f32)
        v_ref[:] = jnp.dot(h2, wv_ref[:], preferred_element_type=f32)

    return pl.pallas_call(
        body,
        grid=(N // R,),
        in_specs=[
            pl.BlockSpec((R, D), lambda i: (i, 0)),
            pl.BlockSpec((NC, R, D), lambda i: (0, i, 0)),
            pl.BlockSpec((R, NC), lambda i: (i, 0)),
            pl.BlockSpec((H, H), lambda i: (0, 0)),
            pl.BlockSpec((H, H), lambda i: (0, 0)),
            pl.BlockSpec((1, H), lambda i: (0, 0)),
            pl.BlockSpec((H, H), lambda i: (0, 0)),
            pl.BlockSpec((H, H), lambda i: (0, 0)),
        ],
        out_specs=[
            pl.BlockSpec((R, H), lambda i: (i, 0)),
            pl.BlockSpec((R, H), lambda i: (i, 0)),
        ],
        out_shape=[
            jax.ShapeDtypeStruct((N, H), f32),
            jax.ShapeDtypeStruct((N, H), f32),
        ],
    )(h1, acc, deg_t, Ws, Wn, br, Wu, Wv)


def _sc_pair_gather_add(u, v, src_r, dst_r):
    mesh = plsc.VectorSubcoreMesh(core_axis_name="c", subcore_axis_name="s")

    @functools.partial(
        pl.kernel,
        out_type=jax.ShapeDtypeStruct((PP, H), f32),
        mesh=mesh,
        scratch_types=[
            pltpu.VMEM((PCH, 1, PC), jnp.int32),
            pltpu.VMEM((PCH, 1, PC), jnp.int32),
            pltpu.VMEM((PC, H), f32),
            pltpu.VMEM((PC, H), f32),
            pltpu.VMEM((PC, H), f32),
            pltpu.VMEM((PC, H), f32),
            pltpu.SemaphoreType.DMA,
            pltpu.SemaphoreType.DMA,
            pltpu.SemaphoreType.DMA,
            pltpu.SemaphoreType.DMA,
        ],
    )
    def k(u_hbm, v_hbm, src_hbm, dst_hbm, z_out, src_v, dst_v,
          u0_v, v0_v, u1_v, v1_v, us0, vs0, us1, vs1):
        cid = lax.axis_index("c")
        sid = lax.axis_index("s")
        wid = cid * NS + sid
        pltpu.sync_copy(src_hbm.at[wid], src_v)
        pltpu.sync_copy(dst_hbm.at[wid], dst_v)
        base = wid * PPW

        def start(j, ub, vb, us, vs):
            pltpu.async_copy(u_hbm.at[src_v.at[j, 0]], ub, us)
            pltpu.async_copy(v_hbm.at[dst_v.at[j, 0]], vb, vs)

        def wait(j, ub, vb, us, vs):
            pltpu.make_async_copy(u_hbm.at[src_v.at[j, 0]], ub, us).wait()
            pltpu.make_async_copy(v_hbm.at[dst_v.at[j, 0]], vb, vs).wait()

        def drain(j, ub, vb):
            def addrow(i, carry):
                for g in range(H // 16):
                    sl = pl.ds(g * 16, 16)
                    ub[i, sl] = ub[i, sl] + vb[i, sl]
                return carry

            lax.fori_loop(0, PC, addrow, 0)
            out_off = pl.multiple_of(base + j * PC, 8)
            pltpu.sync_copy(ub, z_out.at[pl.ds(out_off, PC)])

        start(0, u0_v, v0_v, us0, vs0)

        def chunk2(jj, carry):
            j0 = jj * 2
            j1 = j0 + 1
            start(j1, u1_v, v1_v, us1, vs1)
            wait(j0, u0_v, v0_v, us0, vs0)
            drain(j0, u0_v, v0_v)
            start(jnp.minimum(j1 + 1, PCH - 1), u0_v, v0_v, us0, vs0)
            wait(j1, u1_v, v1_v, us1, vs1)
            drain(j1, u1_v, v1_v)
            return carry

        lax.fori_loop(0, PCH // 2, chunk2, 0)
        wait(PCH - 1, u0_v, v0_v, us0, vs0)

    return k(u, v, src_r, dst_r)


def _tc_mlp(z, b1r, W2, b2r, W3, b3r):
    R = 4096

    def body(z_ref, b1_ref, w2_ref, b2_ref, w3_ref, b3_ref, out_ref):
        z1 = jnp.maximum(z_ref[:] + b1_ref[:], 0.0)
        z2 = jnp.maximum(
            jnp.dot(z1, w2_ref[:], preferred_element_type=f32) + b2_ref[:],
            0.0)
        out_ref[:] = jnp.sum(z2 * w3_ref[:], axis=1, keepdims=True) + b3_ref[:]

    return pl.pallas_call(
        body,
        grid=(PP // R,),
        in_specs=[
            pl.BlockSpec((R, H), lambda i: (i, 0)),
            pl.BlockSpec((1, H), lambda i: (0, 0)),
            pl.BlockSpec((H, H), lambda i: (0, 0)),
            pl.BlockSpec((1, H), lambda i: (0, 0)),
            pl.BlockSpec((1, H), lambda i: (0, 0)),
            pl.BlockSpec((1, 1), lambda i: (0, 0)),
        ],
        out_specs=pl.BlockSpec((R, 1), lambda i: (i, 0)),
        out_shape=jax.ShapeDtypeStruct((PP, 1), f32),
    )(z, b1r, W2, b2r, W3, b3r)


def kernel(x, edge_index1, edge_index2, pos_src, pos_dst, neg_src, neg_dst,
           W_self1, W_neigh1, b1, W_self2, W_neigh2, b2,
           Wp1, bp1, Wp2, bp2, Wp3, bp3):
    zeros2d = jnp.zeros((NPAD, D), f32)
    zeros1d = jnp.zeros((NPAD,), f32)
    pad_src = (jnp.arange(EPAD, dtype=jnp.int32) * 37) % N
    pad_dst = N + jnp.arange(EPAD, dtype=jnp.int32) % (NPAD - N)
    src1 = jnp.concatenate([edge_index1[0], pad_src]).reshape(NW, CH, 1, C)
    dst1 = jnp.concatenate([edge_index1[1], pad_dst]).reshape(NW, CH, 1, C)
    src2 = jnp.concatenate([edge_index2[0], pad_src]).reshape(NW, CH, 1, C)
    dst2 = jnp.concatenate([edge_index2[1], pad_dst]).reshape(NW, CH, 1, C)

    acc1, deg1 = _sc_aggregate(x, src1, dst1, zeros2d, zeros1d)
    h1 = _tc_combine1(x, acc1, deg1.reshape(NC, NPAD).T,
                      W_self1, W_neigh1, b1.reshape(1, H))
    acc2, deg2 = _sc_aggregate(h1, src2, dst2, zeros2d, zeros1d)
    u, v = _tc_combine2(h1, acc2, deg2.reshape(NC, NPAD).T,
                        W_self2, W_neigh2, b2.reshape(1, H),
                        Wp1[:H], Wp1[H:])

    all_src = jnp.concatenate([pos_src, neg_src]).reshape(NW, PCH, 1, PC)
    all_dst = jnp.concatenate([pos_dst, neg_dst]).reshape(NW, PCH, 1, PC)
    z = _sc_pair_gather_add(u, v, all_src, all_dst)

    scores = _tc_mlp(z, bp1.reshape(1, H), Wp2, bp2.reshape(1, H),
                     Wp3.reshape(1, H), bp3.reshape(1, 1))
    return scores[:P], scores[P:]


# speedup vs baseline: 9.2857x; 1.0276x over previous
import functools

import jax
import jax.numpy as jnp
from jax import lax
from jax.experimental import pallas as pl
from jax.experimental.pallas import tpu as pltpu
from jax.experimental.pallas import tpu_sc as plsc

N = 10000
D = 128
H = 128
E = 320000
P = 65536
NC = 2
NS = 16
NW = NC * NS
C = 128
CH = 80
PH = CH // 2
EPAD = NW * CH * C - E
NPAD = 10240
NPT = NPAD // NS
PP = 2 * P
PPW = PP // NW
PC = 128
PCH = PPW // PC

f32 = jnp.float32


def _sc_aggregate(table, src_r, dst_r):
    mesh = plsc.VectorSubcoreMesh(core_axis_name="c", subcore_axis_name="s")

    @functools.partial(
        pl.kernel,
        out_type=(jax.ShapeDtypeStruct((NC, NPAD, D), f32),
                  jax.ShapeDtypeStruct((NC, 1, NPAD), f32)),
        mesh=mesh,
        scratch_types=[
            pltpu.VMEM((PH, 1, C), jnp.int32),
            pltpu.VMEM((PH, 1, C), jnp.int32),
            pltpu.VMEM((C, D), f32),
            pltpu.VMEM((C, D), f32),
            pltpu.VMEM((C,), f32),
            pltpu.VMEM((NPT,), f32),
            pltpu.VMEM_SHARED((NPAD, D), f32),
            pltpu.VMEM_SHARED((NPAD,), f32),
            pltpu.SemaphoreType.DMA,
            pltpu.SemaphoreType.DMA,
            pltpu.SemaphoreType.DMA,
        ],
    )
    def k(tab_hbm, src_hbm, dst_hbm, acc_out, deg_out,
          src_v, dst_v, rows0_v, rows1_v, ones_v, z1_v, acc_sh, deg_sh,
          sem0, sem1, dsem):
        cid = lax.axis_index("c")
        sid = lax.axis_index("s")
        wid = cid * NS + sid
        off = pl.multiple_of(sid * NPT, 128)
        z16 = jnp.zeros((16,), f32)

        def zrow(i, carry):
            for g in range(D // 16):
                rows0_v[i, pl.ds(g * 16, 16)] = z16
            return carry

        lax.fori_loop(0, C, zrow, 0)
        for g in range(NPT // 16):
            z1_v[pl.ds(g * 16, 16)] = z16
        for r in range(NPT // C):
            pltpu.sync_copy(rows0_v, acc_sh.at[pl.ds(off + r * C, C)])
        pltpu.sync_copy(z1_v, deg_sh.at[pl.ds(off, NPT)])

        ones16 = jnp.ones((16,), f32)
        for g in range(C // 16):
            ones_v[pl.ds(g * 16, 16)] = ones16
        plsc.subcore_barrier()

        def start(j, buf, sem):
            pltpu.async_copy(tab_hbm.at[src_v.at[j, 0]], buf, sem)

        def wait(j, buf, sem):
            pltpu.make_async_copy(tab_hbm.at[src_v.at[j, 0]], buf, sem).wait()

        def drain(j, buf):
            dcp = pltpu.async_copy(ones_v, deg_sh.at[dst_v.at[j, 0]], dsem,
                                   add=True)
            pltpu.sync_copy(buf, acc_sh.at[dst_v.at[j, 0]], add=True)
            dcp.wait()

        for p in range(2):
            pltpu.sync_copy(src_hbm.at[wid, pl.ds(p * PH, PH)], src_v)
            pltpu.sync_copy(dst_hbm.at[wid, pl.ds(p * PH, PH)], dst_v)
            start(0, rows0_v, sem0)

            def chunk2(jj, carry):
                j0 = jj * 2
                j1 = j0 + 1
                start(j1, rows1_v, sem1)
                wait(j0, rows0_v, sem0)
                drain(j0, rows0_v)
                start(jnp.minimum(j1 + 1, PH - 1), rows0_v, sem0)
                wait(j1, rows1_v, sem1)
                drain(j1, rows1_v)
                return carry

            lax.fori_loop(0, PH // 2, chunk2, 0)
            wait(PH - 1, rows0_v, sem0)
        plsc.subcore_barrier()
        pltpu.sync_copy(acc_sh.at[pl.ds(off, NPT)],
                        acc_out.at[cid, pl.ds(off, NPT)])
        pltpu.sync_copy(deg_sh.at[pl.ds(off, NPT)],
                        deg_out.at[cid, 0, pl.ds(off, NPT)])

    return k(table, src_r, dst_r)


def _tc_combine1(x, acc, deg_t, Ws, Wn, br):
    R = 2000

    def body(x_ref, acc_ref, deg_ref, ws_ref, wn_ref, b_ref, out_ref):
        accs = acc_ref[0] + acc_ref[1]
        deg = jnp.maximum(jnp.sum(deg_ref[:], axis=1, keepdims=True), 1.0)
        mean = accs / deg
        h = (jnp.dot(x_ref[:], ws_ref[:], preferred_element_type=f32)
             + jnp.dot(mean, wn_ref[:], preferred_element_type=f32)
             + b_ref[:])
        out_ref[:] = jnp.maximum(h, 0.0)

    return pl.pallas_call(
        body,
        grid=(N // R,),
        in_specs=[
            pl.BlockSpec((R, D), lambda i: (i, 0)),
            pl.BlockSpec((NC, R, D), lambda i: (0, i, 0)),
            pl.BlockSpec((R, NC), lambda i: (i, 0)),
            pl.BlockSpec((D, H), lambda i: (0, 0)),
            pl.BlockSpec((D, H), lambda i: (0, 0)),
            pl.BlockSpec((1, H), lambda i: (0, 0)),
        ],
        out_specs=pl.BlockSpec((R, D), lambda i: (i, 0)),
        out_shape=jax.ShapeDtypeStruct((N, D), f32),
    )(x, acc, deg_t, Ws, Wn, br)


def _tc_combine2(h1, acc, deg_t, Ws, Wn, br, Wu, Wv):
    R = 2000

    def body(h_ref, acc_ref, deg_ref, ws_ref, wn_ref, b_ref, wu_ref, wv_ref,
             u_ref, v_ref):
        accs = acc_ref[0] + acc_ref[1]
        deg = jnp.maximum(jnp.sum(deg_ref[:], axis=1, keepdims=True), 1.0)
        mean = accs / deg
        h2 = (jnp.dot(h_ref[:], ws_ref[:], preferred_element_type=f32)
              + jnp.dot(mean, wn_ref[:], preferred_element_type=f32)
              + b_ref[:])
        u_ref[:] = jnp.dot(h2, wu_ref[:], preferred_element_type=f32)
        v_ref[:] = jnp.dot(h2, wv_ref[:], preferred_element_type=f32)

    return pl.pallas_call(
        body,
        grid=(N // R,),
        in_specs=[
            pl.BlockSpec((R, D), lambda i: (i, 0)),
            pl.BlockSpec((NC, R, D), lambda i: (0, i, 0)),
            pl.BlockSpec((R, NC), lambda i: (i, 0)),
            pl.BlockSpec((H, H), lambda i: (0, 0)),
            pl.BlockSpec((H, H), lambda i: (0, 0)),
            pl.BlockSpec((1, H), lambda i: (0, 0)),
            pl.BlockSpec((H, H), lambda i: (0, 0)),
            pl.BlockSpec((H, H), lambda i: (0, 0)),
        ],
        out_specs=[
            pl.BlockSpec((R, H), lambda i: (i, 0)),
            pl.BlockSpec((R, H), lambda i: (i, 0)),
        ],
        out_shape=[
            jax.ShapeDtypeStruct((N, H), f32),
            jax.ShapeDtypeStruct((N, H), f32),
        ],
    )(h1, acc, deg_t, Ws, Wn, br, Wu, Wv)


def _sc_pair_gather_add(u, v, src_r, dst_r):
    mesh = plsc.VectorSubcoreMesh(core_axis_name="c", subcore_axis_name="s")

    @functools.partial(
        pl.kernel,
        out_type=jax.ShapeDtypeStruct((PP, H), f32),
        mesh=mesh,
        scratch_types=[
            pltpu.VMEM((PCH, 1, PC), jnp.int32),
            pltpu.VMEM((PCH, 1, PC), jnp.int32),
            pltpu.VMEM((PC, H), f32),
            pltpu.VMEM((PC, H), f32),
            pltpu.VMEM((PC, H), f32),
            pltpu.VMEM((PC, H), f32),
            pltpu.SemaphoreType.DMA,
            pltpu.SemaphoreType.DMA,
            pltpu.SemaphoreType.DMA,
            pltpu.SemaphoreType.DMA,
        ],
    )
    def k(u_hbm, v_hbm, src_hbm, dst_hbm, z_out, src_v, dst_v,
          u0_v, v0_v, u1_v, v1_v, us0, vs0, us1, vs1):
        cid = lax.axis_index("c")
        sid = lax.axis_index("s")
        wid = cid * NS + sid
        pltpu.sync_copy(src_hbm.at[wid], src_v)
        pltpu.sync_copy(dst_hbm.at[wid], dst_v)
        base = wid * PPW

        def start(j, ub, vb, us, vs):
            pltpu.async_copy(u_hbm.at[src_v.at[j, 0]], ub, us)
            pltpu.async_copy(v_hbm.at[dst_v.at[j, 0]], vb, vs)

        def wait(j, ub, vb, us, vs):
            pltpu.make_async_copy(u_hbm.at[src_v.at[j, 0]], ub, us).wait()
            pltpu.make_async_copy(v_hbm.at[dst_v.at[j, 0]], vb, vs).wait()

        def drain(j, ub, vb):
            def addrow(i, carry):
                for g in range(H // 16):
                    sl = pl.ds(g * 16, 16)
                    ub[i, sl] = ub[i, sl] + vb[i, sl]
                return carry

            lax.fori_loop(0, PC, addrow, 0)
            out_off = pl.multiple_of(base + j * PC, 8)
            pltpu.sync_copy(ub, z_out.at[pl.ds(out_off, PC)])

        start(0, u0_v, v0_v, us0, vs0)

        def chunk2(jj, carry):
            j0 = jj * 2
            j1 = j0 + 1
            start(j1, u1_v, v1_v, us1, vs1)
            wait(j0, u0_v, v0_v, us0, vs0)
            drain(j0, u0_v, v0_v)
            start(jnp.minimum(j1 + 1, PCH - 1), u0_v, v0_v, us0, vs0)
            wait(j1, u1_v, v1_v, us1, vs1)
            drain(j1, u1_v, v1_v)
            return carry

        lax.fori_loop(0, PCH // 2, chunk2, 0)
        wait(PCH - 1, u0_v, v0_v, us0, vs0)

    return k(u, v, src_r, dst_r)


def _tc_mlp(z, b1r, W2, b2r, W3, b3r):
    R = 4096

    def body(z_ref, b1_ref, w2_ref, b2_ref, w3_ref, b3_ref, out_ref):
        z1 = jnp.maximum(z_ref[:] + b1_ref[:], 0.0)
        z2 = jnp.maximum(
            jnp.dot(z1, w2_ref[:], preferred_element_type=f32) + b2_ref[:],
            0.0)
        out_ref[:] = jnp.sum(z2 * w3_ref[:], axis=1, keepdims=True) + b3_ref[:]

    return pl.pallas_call(
        body,
        grid=(PP // R,),
        in_specs=[
            pl.BlockSpec((R, H), lambda i: (i, 0)),
            pl.BlockSpec((1, H), lambda i: (0, 0)),
            pl.BlockSpec((H, H), lambda i: (0, 0)),
            pl.BlockSpec((1, H), lambda i: (0, 0)),
            pl.BlockSpec((1, H), lambda i: (0, 0)),
            pl.BlockSpec((1, 1), lambda i: (0, 0)),
        ],
        out_specs=pl.BlockSpec((R, 1), lambda i: (i, 0)),
        out_shape=jax.ShapeDtypeStruct((PP, 1), f32),
    )(z, b1r, W2, b2r, W3, b3r)


def kernel(x, edge_index1, edge_index2, pos_src, pos_dst, neg_src, neg_dst,
           W_self1, W_neigh1, b1, W_self2, W_neigh2, b2,
           Wp1, bp1, Wp2, bp2, Wp3, bp3):
    pad_src = (jnp.arange(EPAD, dtype=jnp.int32) * 37) % N
    pad_dst = N + jnp.arange(EPAD, dtype=jnp.int32) % (NPAD - N)
    src1 = jnp.concatenate([edge_index1[0], pad_src]).reshape(NW, CH, 1, C)
    dst1 = jnp.concatenate([edge_index1[1], pad_dst]).reshape(NW, CH, 1, C)
    src2 = jnp.concatenate([edge_index2[0], pad_src]).reshape(NW, CH, 1, C)
    dst2 = jnp.concatenate([edge_index2[1], pad_dst]).reshape(NW, CH, 1, C)

    acc1, deg1 = _sc_aggregate(x, src1, dst1)
    h1 = _tc_combine1(x, acc1, deg1.reshape(NC, NPAD).T,
                      W_self1, W_neigh1, b1.reshape(1, H))
    acc2, deg2 = _sc_aggregate(h1, src2, dst2)
    u, v = _tc_combine2(h1, acc2, deg2.reshape(NC, NPAD).T,
                        W_self2, W_neigh2, b2.reshape(1, H),
                        Wp1[:H], Wp1[H:])

    all_src = jnp.concatenate([pos_src, neg_src]).reshape(NW, PCH, 1, PC)
    all_dst = jnp.concatenate([pos_dst, neg_dst]).reshape(NW, PCH, 1, PC)
    z = _sc_pair_gather_add(u, v, all_src, all_dst)

    scores = _tc_mlp(z, bp1.reshape(1, H), Wp2, bp2.reshape(1, H),
                     Wp3.reshape(1, H), bp3.reshape(1, 1))
    return scores[:P], scores[P:]


# speedup vs baseline: 10.3019x; 1.1094x over previous
import functools

import jax
import jax.numpy as jnp
from jax import lax
from jax.experimental import pallas as pl
from jax.experimental.pallas import tpu as pltpu
from jax.experimental.pallas import tpu_sc as plsc

N = 10000
D = 128
H = 128
E = 320000
P = 65536
NC = 2
NS = 16
NW = NC * NS
C = 128
CH = 80
PH = CH // 2
EPAD = NW * CH * C - E
NPAD = 10240
NPT = NPAD // NS
PP = 2 * P
PPW = PP // NW
PC = 128
PCH = PPW // PC

f32 = jnp.float32


def _sc_aggregate(table, src_r, dst_r):
    mesh = plsc.VectorSubcoreMesh(core_axis_name="c", subcore_axis_name="s")

    @functools.partial(
        pl.kernel,
        out_type=(jax.ShapeDtypeStruct((NC, NPAD, D), f32),
                  jax.ShapeDtypeStruct((NC, 1, NPAD), f32)),
        mesh=mesh,
        scratch_types=[
            pltpu.VMEM((PH, 1, C), jnp.int32),
            pltpu.VMEM((PH, 1, C), jnp.int32),
            pltpu.VMEM((C, D), f32),
            pltpu.VMEM((C, D), f32),
            pltpu.VMEM((C,), f32),
            pltpu.VMEM((NPT,), f32),
            pltpu.VMEM_SHARED((NPAD, D), f32),
            pltpu.VMEM_SHARED((NPAD,), f32),
            pltpu.SemaphoreType.DMA,
            pltpu.SemaphoreType.DMA,
            pltpu.SemaphoreType.DMA,
        ],
    )
    def k(tab_hbm, src_hbm, dst_hbm, acc_out, deg_out,
          src_v, dst_v, rows0_v, rows1_v, ones_v, z1_v, acc_sh, deg_sh,
          sem0, sem1, dsem):
        cid = lax.axis_index("c")
        sid = lax.axis_index("s")
        wid = cid * NS + sid
        off = pl.multiple_of(sid * NPT, 128)
        z16 = jnp.zeros((16,), f32)

        def zrow(i, carry):
            for g in range(D // 16):
                rows0_v[i, pl.ds(g * 16, 16)] = z16
            return carry

        lax.fori_loop(0, C, zrow, 0)
        for g in range(NPT // 16):
            z1_v[pl.ds(g * 16, 16)] = z16
        for r in range(NPT // C):
            pltpu.sync_copy(rows0_v, acc_sh.at[pl.ds(off + r * C, C)])
        pltpu.sync_copy(z1_v, deg_sh.at[pl.ds(off, NPT)])

        ones16 = jnp.ones((16,), f32)
        for g in range(C // 16):
            ones_v[pl.ds(g * 16, 16)] = ones16
        plsc.subcore_barrier()

        def start(j, buf, sem):
            pltpu.async_copy(tab_hbm.at[src_v.at[j, 0]], buf, sem)

        def wait(j, buf, sem):
            pltpu.make_async_copy(tab_hbm.at[src_v.at[j, 0]], buf, sem).wait()

        def drain(j, buf):
            dcp = pltpu.async_copy(ones_v, deg_sh.at[dst_v.at[j, 0]], dsem,
                                   add=True)
            pltpu.sync_copy(buf, acc_sh.at[dst_v.at[j, 0]], add=True)
            dcp.wait()

        for p in range(2):
            pltpu.sync_copy(src_hbm.at[wid, pl.ds(p * PH, PH)], src_v)
            pltpu.sync_copy(dst_hbm.at[wid, pl.ds(p * PH, PH)], dst_v)
            start(0, rows0_v, sem0)

            def chunk2(jj, carry):
                j0 = jj * 2
                j1 = j0 + 1
                start(j1, rows1_v, sem1)
                wait(j0, rows0_v, sem0)
                drain(j0, rows0_v)
                start(jnp.minimum(j1 + 1, PH - 1), rows0_v, sem0)
                wait(j1, rows1_v, sem1)
                drain(j1, rows1_v)
                return carry

            lax.fori_loop(0, PH // 2, chunk2, 0)
            wait(PH - 1, rows0_v, sem0)
        plsc.subcore_barrier()
        pltpu.sync_copy(acc_sh.at[pl.ds(off, NPT)],
                        acc_out.at[cid, pl.ds(off, NPT)])
        pltpu.sync_copy(deg_sh.at[pl.ds(off, NPT)],
                        deg_out.at[cid, 0, pl.ds(off, NPT)])

    return k(table, src_r, dst_r)


def _tc_combine1(x, acc, deg_t, Ws, Wn, br):
    R = 2000

    def body(x_ref, acc_ref, deg_ref, ws_ref, wn_ref, b_ref, out_ref):
        accs = acc_ref[0] + acc_ref[1]
        deg = jnp.maximum(jnp.sum(deg_ref[:], axis=1, keepdims=True), 1.0)
        mean = accs / deg
        h = (jnp.dot(x_ref[:], ws_ref[:], preferred_element_type=f32)
             + jnp.dot(mean, wn_ref[:], preferred_element_type=f32)
             + b_ref[:])
        out_ref[:] = jnp.maximum(h, 0.0)

    return pl.pallas_call(
        body,
        grid=(N // R,),
        in_specs=[
            pl.BlockSpec((R, D), lambda i: (i, 0)),
            pl.BlockSpec((NC, R, D), lambda i: (0, i, 0)),
            pl.BlockSpec((R, NC), lambda i: (i, 0)),
            pl.BlockSpec((D, H), lambda i: (0, 0)),
            pl.BlockSpec((D, H), lambda i: (0, 0)),
            pl.BlockSpec((1, H), lambda i: (0, 0)),
        ],
        out_specs=pl.BlockSpec((R, D), lambda i: (i, 0)),
        out_shape=jax.ShapeDtypeStruct((N, D), f32),
    )(x, acc, deg_t, Ws, Wn, br)


def _tc_combine2(h1, acc, deg_t, Ws, Wn, br, Wu, Wv):
    R = 2000

    def body(h_ref, acc_ref, deg_ref, ws_ref, wn_ref, b_ref, wu_ref, wv_ref,
             u_ref, v_ref):
        accs = acc_ref[0] + acc_ref[1]
        deg = jnp.maximum(jnp.sum(deg_ref[:], axis=1, keepdims=True), 1.0)
        mean = accs / deg
        h2 = (jnp.dot(h_ref[:], ws_ref[:], preferred_element_type=f32)
              + jnp.dot(mean, wn_ref[:], preferred_element_type=f32)
              + b_ref[:])
        u_ref[:] = jnp.dot(h2, wu_ref[:], preferred_element_type=f32)
        v_ref[:] = jnp.dot(h2, wv_ref[:], preferred_element_type=f32)

    return pl.pallas_call(
        body,
        grid=(N // R,),
        in_specs=[
            pl.BlockSpec((R, D), lambda i: (i, 0)),
            pl.BlockSpec((NC, R, D), lambda i: (0, i, 0)),
            pl.BlockSpec((R, NC), lambda i: (i, 0)),
            pl.BlockSpec((H, H), lambda i: (0, 0)),
            pl.BlockSpec((H, H), lambda i: (0, 0)),
            pl.BlockSpec((1, H), lambda i: (0, 0)),
            pl.BlockSpec((H, H), lambda i: (0, 0)),
            pl.BlockSpec((H, H), lambda i: (0, 0)),
        ],
        out_specs=[
            pl.BlockSpec((R, H), lambda i: (i, 0)),
            pl.BlockSpec((R, H), lambda i: (i, 0)),
        ],
        out_shape=[
            jax.ShapeDtypeStruct((N, H), f32),
            jax.ShapeDtypeStruct((N, H), f32),
        ],
    )(h1, acc, deg_t, Ws, Wn, br, Wu, Wv)


def _sc_pair_gather_add(u, v, src_ids, dst_ids, npairs):
    W = npairs // NW
    CHp = W // PC
    mesh = plsc.VectorSubcoreMesh(core_axis_name="c", subcore_axis_name="s")

    @functools.partial(
        pl.kernel,
        out_type=jax.ShapeDtypeStruct((npairs, H), f32),
        mesh=mesh,
        scratch_types=[
            pltpu.VMEM((W,), jnp.int32),
            pltpu.VMEM((W,), jnp.int32),
            pltpu.VMEM((PC, H), f32),
            pltpu.VMEM((PC, H), f32),
            pltpu.VMEM((PC, H), f32),
            pltpu.VMEM((PC, H), f32),
            pltpu.SemaphoreType.DMA,
            pltpu.SemaphoreType.DMA,
            pltpu.SemaphoreType.DMA,
            pltpu.SemaphoreType.DMA,
        ],
    )
    def k(u_hbm, v_hbm, src_hbm, dst_hbm, z_out, src_v, dst_v,
          u0_v, v0_v, u1_v, v1_v, us0, vs0, us1, vs1):
        cid = lax.axis_index("c")
        sid = lax.axis_index("s")
        wid = cid * NS + sid
        base = pl.multiple_of(wid * W, 128)
        pltpu.sync_copy(src_hbm.at[pl.ds(base, W)], src_v)
        pltpu.sync_copy(dst_hbm.at[pl.ds(base, W)], dst_v)

        def idx(ref, j):
            return ref.at[pl.ds(pl.multiple_of(j * PC, 128), PC)]

        def start(j, ub, vb, us, vs):
            pltpu.async_copy(u_hbm.at[idx(src_v, j)], ub, us)
            pltpu.async_copy(v_hbm.at[idx(dst_v, j)], vb, vs)

        def wait(j, ub, vb, us, vs):
            pltpu.make_async_copy(u_hbm.at[idx(src_v, j)], ub, us).wait()
            pltpu.make_async_copy(v_hbm.at[idx(dst_v, j)], vb, vs).wait()

        def drain(j, ub, vb):
            def addrow(i, carry):
                for g in range(H // 16):
                    sl = pl.ds(g * 16, 16)
                    ub[i, sl] = ub[i, sl] + vb[i, sl]
                return carry

            lax.fori_loop(0, PC, addrow, 0)
            out_off = pl.multiple_of(base + j * PC, 8)
            pltpu.sync_copy(ub, z_out.at[pl.ds(out_off, PC)])

        start(0, u0_v, v0_v, us0, vs0)

        def chunk2(jj, carry):
            j0 = jj * 2
            j1 = j0 + 1
            start(j1, u1_v, v1_v, us1, vs1)
            wait(j0, u0_v, v0_v, us0, vs0)
            drain(j0, u0_v, v0_v)
            start(jnp.minimum(j1 + 1, CHp - 1), u0_v, v0_v, us0, vs0)
            wait(j1, u1_v, v1_v, us1, vs1)
            drain(j1, u1_v, v1_v)
            return carry

        lax.fori_loop(0, CHp // 2, chunk2, 0)
        wait(CHp - 1, u0_v, v0_v, us0, vs0)

    return k(u, v, src_ids, dst_ids)


def _tc_mlp(z, b1r, W2, b2r, W3, b3r):
    R = 4096
    npairs = z.shape[0]

    def body(z_ref, b1_ref, w2_ref, b2_ref, w3_ref, b3_ref, out_ref):
        z1 = jnp.maximum(z_ref[:] + b1_ref[:], 0.0)
        z2 = jnp.maximum(
            jnp.dot(z1, w2_ref[:], preferred_element_type=f32) + b2_ref[:],
            0.0)
        out_ref[:] = jnp.sum(z2 * w3_ref[:], axis=1, keepdims=True) + b3_ref[:]

    return pl.pallas_call(
        body,
        grid=(npairs // R,),
        in_specs=[
            pl.BlockSpec((R, H), lambda i: (i, 0)),
            pl.BlockSpec((1, H), lambda i: (0, 0)),
            pl.BlockSpec((H, H), lambda i: (0, 0)),
            pl.BlockSpec((1, H), lambda i: (0, 0)),
            pl.BlockSpec((1, H), lambda i: (0, 0)),
            pl.BlockSpec((1, 1), lambda i: (0, 0)),
        ],
        out_specs=pl.BlockSpec((R, 1), lambda i: (i, 0)),
        out_shape=jax.ShapeDtypeStruct((npairs, 1), f32),
    )(z, b1r, W2, b2r, W3, b3r)


def kernel(x, edge_index1, edge_index2, pos_src, pos_dst, neg_src, neg_dst,
           W_self1, W_neigh1, b1, W_self2, W_neigh2, b2,
           Wp1, bp1, Wp2, bp2, Wp3, bp3):
    pad_src = (jnp.arange(EPAD, dtype=jnp.int32) * 37) % N
    pad_dst = N + jnp.arange(EPAD, dtype=jnp.int32) % (NPAD - N)
    src1 = jnp.concatenate([edge_index1[0], pad_src]).reshape(NW, CH, 1, C)
    dst1 = jnp.concatenate([edge_index1[1], pad_dst]).reshape(NW, CH, 1, C)
    src2 = jnp.concatenate([edge_index2[0], pad_src]).reshape(NW, CH, 1, C)
    dst2 = jnp.concatenate([edge_index2[1], pad_dst]).reshape(NW, CH, 1, C)

    acc1, deg1 = _sc_aggregate(x, src1, dst1)
    h1 = _tc_combine1(x, acc1, deg1.reshape(NC, NPAD).T,
                      W_self1, W_neigh1, b1.reshape(1, H))
    acc2, deg2 = _sc_aggregate(h1, src2, dst2)
    u, v = _tc_combine2(h1, acc2, deg2.reshape(NC, NPAD).T,
                        W_self2, W_neigh2, b2.reshape(1, H),
                        Wp1[:H], Wp1[H:])

    b1r, b2r = bp1.reshape(1, H), bp2.reshape(1, H)
    w3r, b3r = Wp3.reshape(1, H), bp3.reshape(1, 1)
    z_pos = _sc_pair_gather_add(u, v, pos_src, pos_dst, P)
    z_neg = _sc_pair_gather_add(u, v, neg_src, neg_dst, P)
    s_pos = _tc_mlp(z_pos, b1r, Wp2, b2r, w3r, b3r)
    s_neg = _tc_mlp(z_neg, b1r, Wp2, b2r, w3r, b3r)
    return s_pos, s_neg
